# baseline (speedup 1.0000x reference)
import functools

import jax
import jax.numpy as jnp
from jax import lax
from jax.experimental import pallas as pl
from jax.experimental.pallas import tpu as pltpu
from jax.experimental.pallas import tpu_sc as plsc

D_MODEL = 2048
D_LATENT = 512
D_HIDDEN = 1024
NUM_EXPERTS = 8
TOP_K = 2
N_TOKENS = 4096
TILE = 256
TILE_A = 2048
TILE_E = 1024
TILE_G = 256
NT = N_TOKENS * TOP_K // TILE_G + NUM_EXPERTS - 1
S_PAD = NT * TILE_G

_NEG_BIG = -3.0e38


def _router_z_body(x_ref, wr_ref, wd_ref, z_ref,
                   topi0_ref, topi1_ref, gates0_ref, gates1_ref):
    x = x_ref[...]
    logits = jnp.dot(x, wr_ref[...], preferred_element_type=jnp.float32)
    col = lax.broadcasted_iota(jnp.int32, logits.shape, 1)
    m1 = jnp.max(logits, axis=-1, keepdims=True)
    i1 = jnp.min(jnp.where(logits == m1, col, NUM_EXPERTS), axis=-1, keepdims=True)
    masked = jnp.where(col == i1, _NEG_BIG, logits)
    m2 = jnp.max(masked, axis=-1, keepdims=True)
    i2 = jnp.min(jnp.where(masked == m2, col, NUM_EXPERTS), axis=-1, keepdims=True)
    g0 = 1.0 / (1.0 + jnp.exp(m2 - m1))
    topi0_ref[...] = i1
    topi1_ref[...] = i2
    gates0_ref[...] = g0
    gates1_ref[...] = 1.0 - g0
    z_ref[...] = jnp.dot(x.astype(jnp.bfloat16),
                         wd_ref[...].astype(jnp.bfloat16),
                         preferred_element_type=jnp.float32)


_MR = 64
_MC = 128


def _metadata_body(eid0_ref, eid1_ref, slot0_ref, slot1_ref, tile_eid_ref):
    eid = jnp.concatenate([eid0_ref[...], eid1_ref[...]], axis=0)
    row = lax.broadcasted_iota(jnp.int32, (_MR, _MR), 0)
    rowp = lax.broadcasted_iota(jnp.int32, (_MR, _MR), 1)
    l_strict = (rowp < row).astype(jnp.float32)
    colp = lax.broadcasted_iota(jnp.int32, (_MC, _MC), 0)
    colc = lax.broadcasted_iota(jnp.int32, (_MC, _MC), 1)
    u_strict = (colp < colc).astype(jnp.float32)
    ones_c = jnp.ones((_MC, _MC), jnp.float32)

    slot_f = jnp.zeros((_MR, _MC), jnp.float32)
    off = 0
    starts = []
    ntiles = []
    for e in range(NUM_EXPERTS):
        m_e = (eid == e).astype(jnp.float32)
        prior_rows = jnp.dot(
            jnp.dot(l_strict, m_e, preferred_element_type=jnp.float32),
            ones_c, preferred_element_type=jnp.float32)
        in_row = jnp.dot(m_e, u_strict, preferred_element_type=jnp.float32)
        rank = prior_rows + in_row
        c_e = jnp.sum(m_e).astype(jnp.int32)
        off_f = off.astype(jnp.float32) if e > 0 else jnp.float32(0.0)
        slot_f = slot_f + m_e * (off_f + rank)
        nt_e = (c_e + (TILE_G - 1)) // TILE_G
        start_e = (off // TILE_G) if e > 0 else jnp.int32(0)
        starts.append(start_e)
        ntiles.append(nt_e)
        off = (off if e > 0 else jnp.int32(0)) + nt_e * TILE_G
    slot = slot_f.astype(jnp.int32)
    slot0_ref[...] = slot[:_MR // 2, :]
    slot1_ref[...] = slot[_MR // 2:, :]

    tcol = lax.broadcasted_iota(jnp.int32, (8, 128), 1)
    acc = jnp.zeros((8, 128), jnp.int32)
    for e in range(NUM_EXPERTS):
        hit = (tcol >= starts[e]) & (tcol < starts[e] + ntiles[e])
        acc = acc + e * hit.astype(jnp.int32)
    tile_eid_ref[...] = acc


TPW = N_TOKENS // 32


def _sc_dispatch_body(z_hbm, s0_hbm, s1_hbm, zs_hbm, i0_v, i1_v, rows_v, sem):
    wid = lax.axis_index("s") * 2 + lax.axis_index("c")
    base = wid * TPW
    pltpu.sync_copy(s0_hbm.at[pl.ds(base, TPW)], i0_v)
    pltpu.sync_copy(s1_hbm.at[pl.ds(base, TPW)], i1_v)
    pltpu.sync_copy(z_hbm.at[pl.ds(base, TPW)], rows_v)
    pltpu.async_copy(rows_v, zs_hbm.at[i0_v], sem).wait()
    pltpu.async_copy(rows_v, zs_hbm.at[i1_v], sem).wait()


def _sc_combine_body(os_hbm, s0_hbm, s1_hbm, y0_hbm, y1_hbm,
                     i_v, rows_v, sem):
    wid = lax.axis_index("s") * 2 + lax.axis_index("c")
    base = wid * TPW
    pltpu.sync_copy(s0_hbm.at[pl.ds(base, TPW)], i_v)
    pltpu.async_copy(os_hbm.at[i_v], rows_v, sem).wait()
    pltpu.sync_copy(rows_v, y0_hbm.at[pl.ds(base, TPW)])
    pltpu.sync_copy(s1_hbm.at[pl.ds(base, TPW)], i_v)
    pltpu.async_copy(os_hbm.at[i_v], rows_v, sem).wait()
    pltpu.sync_copy(rows_v, y1_hbm.at[pl.ds(base, TPW)])


def _group_mm_body(eids_ref, zs_ref, wg_ref, w1_ref, w2_ref, out_ref):
    z = zs_ref[...].astype(jnp.bfloat16)
    g = jnp.dot(z, wg_ref[0].astype(jnp.bfloat16),
                preferred_element_type=jnp.float32)
    a = jnp.dot(z, w1_ref[0].astype(jnp.bfloat16),
                preferred_element_type=jnp.float32)
    h = (g / (1.0 + jnp.exp(-g))) * a
    out_ref[...] = jnp.dot(h.astype(jnp.bfloat16),
                           w2_ref[0].astype(jnp.bfloat16),
                           preferred_element_type=jnp.float32)


def _combine_up_body(y0_ref, y1_ref, g0_ref, g1_ref, wu_ref, y_ref):
    y_lat = g0_ref[...] * y0_ref[...] + g1_ref[...] * y1_ref[...]
    y_ref[...] = jnp.dot(y_lat.astype(jnp.bfloat16),
                         wu_ref[...].astype(jnp.bfloat16),
                         preferred_element_type=jnp.float32)


@jax.jit
def kernel(x, Wr, Wd, Wu, w1, wg, w2):
    B, S, D = x.shape
    N = B * S
    t = x.reshape(N, D)
    n_tiles = N // TILE

    z, topi0, topi1, gates0, gates1 = pl.pallas_call(
        _router_z_body,
        grid=(N // TILE_A,),
        in_specs=[
            pl.BlockSpec((TILE_A, D_MODEL), lambda i: (i, 0)),
            pl.BlockSpec((D_MODEL, NUM_EXPERTS), lambda i: (0, 0)),
            pl.BlockSpec((D_MODEL, D_LATENT), lambda i: (0, 0)),
        ],
        out_specs=[
            pl.BlockSpec((TILE_A, D_LATENT), lambda i: (i, 0)),
            pl.BlockSpec((TILE_A, 1), lambda i: (i, 0)),
            pl.BlockSpec((TILE_A, 1), lambda i: (i, 0)),
            pl.BlockSpec((TILE_A, 1), lambda i: (i, 0)),
            pl.BlockSpec((TILE_A, 1), lambda i: (i, 0)),
        ],
        out_shape=[
            jax.ShapeDtypeStruct((N, D_LATENT), jnp.float32),
            jax.ShapeDtypeStruct((N, 1), jnp.int32),
            jax.ShapeDtypeStruct((N, 1), jnp.int32),
            jax.ShapeDtypeStruct((N, 1), jnp.float32),
            jax.ShapeDtypeStruct((N, 1), jnp.float32),
        ],
    )(t, Wr, Wd)

    slot0m, slot1m, tile_eid8 = pl.pallas_call(
        _metadata_body,
        grid=(1,),
        in_specs=[
            pl.BlockSpec((_MR // 2, _MC), lambda i: (0, 0)),
            pl.BlockSpec((_MR // 2, _MC), lambda i: (0, 0)),
        ],
        out_specs=[
            pl.BlockSpec((_MR // 2, _MC), lambda i: (0, 0)),
            pl.BlockSpec((_MR // 2, _MC), lambda i: (0, 0)),
            pl.BlockSpec((8, 128), lambda i: (0, 0)),
        ],
        out_shape=[
            jax.ShapeDtypeStruct((_MR // 2, _MC), jnp.int32),
            jax.ShapeDtypeStruct((_MR // 2, _MC), jnp.int32),
            jax.ShapeDtypeStruct((8, 128), jnp.int32),
        ],
    )(topi0.reshape(_MR // 2, _MC), topi1.reshape(_MR // 2, _MC))

    slot0 = slot0m.reshape(N)
    slot1 = slot1m.reshape(N)

    sc_mesh = plsc.VectorSubcoreMesh(core_axis_name="c", subcore_axis_name="s")

    z_sorted = pl.kernel(
        _sc_dispatch_body,
        out_type=jax.ShapeDtypeStruct((S_PAD, D_LATENT), jnp.float32),
        mesh=sc_mesh,
        scratch_types=[
            pltpu.VMEM((TPW,), jnp.int32),
            pltpu.VMEM((TPW,), jnp.int32),
            pltpu.VMEM((TPW, D_LATENT), jnp.float32),
            pltpu.SemaphoreType.DMA,
        ],
    )(z, slot0, slot1)

    out_sorted = pl.pallas_call(
        _group_mm_body,
        grid_spec=pltpu.PrefetchScalarGridSpec(
            num_scalar_prefetch=1,
            grid=(NT,),
            in_specs=[
                pl.BlockSpec((TILE_G, D_LATENT), lambda i, eids: (i, 0)),
                pl.BlockSpec((1, D_LATENT, D_HIDDEN),
                             lambda i, eids: (eids[0, i], 0, 0)),
                pl.BlockSpec((1, D_LATENT, D_HIDDEN),
                             lambda i, eids: (eids[0, i], 0, 0)),
                pl.BlockSpec((1, D_HIDDEN, D_LATENT),
                             lambda i, eids: (eids[0, i], 0, 0)),
            ],
            out_specs=pl.BlockSpec((TILE_G, D_LATENT), lambda i, eids: (i, 0)),
        ),
        out_shape=jax.ShapeDtypeStruct((S_PAD, D_LATENT), jnp.float32),
    )(tile_eid8, z_sorted, wg, w1, w2)

    y0, y1 = pl.kernel(
        _sc_combine_body,
        out_type=[
            jax.ShapeDtypeStruct((N, D_LATENT), jnp.float32),
            jax.ShapeDtypeStruct((N, D_LATENT), jnp.float32),
        ],
        mesh=sc_mesh,
        scratch_types=[
            pltpu.VMEM((TPW,), jnp.int32),
            pltpu.VMEM((TPW, D_LATENT), jnp.float32),
            pltpu.SemaphoreType.DMA,
        ],
    )(out_sorted, slot0, slot1)

    y = pl.pallas_call(
        _combine_up_body,
        grid=(N // TILE_E,),
        in_specs=[
            pl.BlockSpec((TILE_E, D_LATENT), lambda i: (i, 0)),
            pl.BlockSpec((TILE_E, D_LATENT), lambda i: (i, 0)),
            pl.BlockSpec((TILE_E, 1), lambda i: (i, 0)),
            pl.BlockSpec((TILE_E, 1), lambda i: (i, 0)),
            pl.BlockSpec((D_LATENT, D_MODEL), lambda i: (0, 0)),
        ],
        out_specs=pl.BlockSpec((TILE_E, D_MODEL), lambda i: (i, 0)),
        out_shape=jax.ShapeDtypeStruct((N, D_MODEL), jnp.float32),
    )(y0, y1, gates0, gates1, Wu)

    return y.reshape(B, S, D)


# speedup vs baseline: 1.0696x; 1.0696x over previous
import functools

import jax
import jax.numpy as jnp
from jax import lax
from jax.experimental import pallas as pl
from jax.experimental.pallas import tpu as pltpu
from jax.experimental.pallas import tpu_sc as plsc

D_MODEL = 2048
D_LATENT = 512
D_HIDDEN = 1024
NUM_EXPERTS = 8
TOP_K = 2
N_TOKENS = 4096
TILE = 256
TILE_A = 1024
TILE_E = 512
TILE_G = 256
NT = N_TOKENS * TOP_K // TILE_G + NUM_EXPERTS - 1
S_PAD = NT * TILE_G

_NEG_BIG = -3.0e38


def _router_z_body(x_ref, wr_ref, wd_ref, z_ref,
                   topi0_ref, topi1_ref, gates0_ref, gates1_ref):
    x = x_ref[...]
    logits = jnp.dot(x, wr_ref[...], preferred_element_type=jnp.float32)
    col = lax.broadcasted_iota(jnp.int32, logits.shape, 1)
    m1 = jnp.max(logits, axis=-1, keepdims=True)
    i1 = jnp.min(jnp.where(logits == m1, col, NUM_EXPERTS), axis=-1, keepdims=True)
    masked = jnp.where(col == i1, _NEG_BIG, logits)
    m2 = jnp.max(masked, axis=-1, keepdims=True)
    i2 = jnp.min(jnp.where(masked == m2, col, NUM_EXPERTS), axis=-1, keepdims=True)
    g0 = 1.0 / (1.0 + jnp.exp(m2 - m1))
    topi0_ref[...] = i1
    topi1_ref[...] = i2
    gates0_ref[...] = g0
    gates1_ref[...] = 1.0 - g0
    z_ref[...] = jnp.dot(x.astype(jnp.bfloat16),
                         wd_ref[...].astype(jnp.bfloat16),
                         preferred_element_type=jnp.float32)


_MR = 64
_MC = 128


def _metadata_body(eid0_ref, eid1_ref, slot0_ref, slot1_ref, tile_eid_ref):
    eid = jnp.concatenate([eid0_ref[...], eid1_ref[...]], axis=0)
    row = lax.broadcasted_iota(jnp.int32, (_MR, _MR), 0)
    rowp = lax.broadcasted_iota(jnp.int32, (_MR, _MR), 1)
    l_strict = (rowp < row).astype(jnp.float32)
    colp = lax.broadcasted_iota(jnp.int32, (_MC, _MC), 0)
    colc = lax.broadcasted_iota(jnp.int32, (_MC, _MC), 1)
    u_strict = (colp < colc).astype(jnp.float32)
    ones_c = jnp.ones((_MC, _MC), jnp.float32)

    slot_f = jnp.zeros((_MR, _MC), jnp.float32)
    off = 0
    starts = []
    ntiles = []
    for e in range(NUM_EXPERTS):
        m_e = (eid == e).astype(jnp.float32)
        prior_rows = jnp.dot(
            jnp.dot(l_strict, m_e, preferred_element_type=jnp.float32),
            ones_c, preferred_element_type=jnp.float32)
        in_row = jnp.dot(m_e, u_strict, preferred_element_type=jnp.float32)
        rank = prior_rows + in_row
        c_e = jnp.sum(m_e).astype(jnp.int32)
        off_f = off.astype(jnp.float32) if e > 0 else jnp.float32(0.0)
        slot_f = slot_f + m_e * (off_f + rank)
        nt_e = (c_e + (TILE_G - 1)) // TILE_G
        start_e = (off // TILE_G) if e > 0 else jnp.int32(0)
        starts.append(start_e)
        ntiles.append(nt_e)
        off = (off if e > 0 else jnp.int32(0)) + nt_e * TILE_G
    slot = slot_f.astype(jnp.int32)
    slot0_ref[...] = slot[:_MR // 2, :]
    slot1_ref[...] = slot[_MR // 2:, :]

    tcol = lax.broadcasted_iota(jnp.int32, (8, 128), 1)
    acc = jnp.zeros((8, 128), jnp.int32)
    acc_run = jnp.zeros((8, 128), jnp.int32)
    acc_reid = jnp.zeros((8, 128), jnp.int32)
    prefix_present = jnp.int32(0)
    for e in range(NUM_EXPERTS):
        hit = ((tcol >= starts[e]) & (tcol < starts[e] + ntiles[e])
               ).astype(jnp.int32)
        acc = acc + e * hit
        acc_run = acc_run + prefix_present * hit
        present = (ntiles[e] > 0).astype(jnp.int32)
        acc_reid = acc_reid + e * ((tcol == prefix_present).astype(jnp.int32)
                                   * present)
        prefix_present = prefix_present + present
    total_used = off // TILE_G
    acc_run = acc_run + (prefix_present - 1) * (tcol >= total_used).astype(
        jnp.int32)
    tile_eid_ref[...] = jnp.concatenate(
        [acc[0:1, :], acc_run[0:1, :], acc_reid[0:1, :], acc[0:5, :]], axis=0)


TPW = N_TOKENS // 32


def _sc_dispatch_body(z_hbm, s0_hbm, s1_hbm, zs_hbm, i0_v, i1_v, rows_v, sem):
    wid = lax.axis_index("s") * 2 + lax.axis_index("c")
    base = wid * TPW
    pltpu.sync_copy(s0_hbm.at[pl.ds(base, TPW)], i0_v)
    pltpu.sync_copy(s1_hbm.at[pl.ds(base, TPW)], i1_v)
    pltpu.sync_copy(z_hbm.at[pl.ds(base, TPW)], rows_v)
    pltpu.async_copy(rows_v, zs_hbm.at[i0_v], sem).wait()
    pltpu.async_copy(rows_v, zs_hbm.at[i1_v], sem).wait()


def _sc_combine_body(os_hbm, s0_hbm, s1_hbm, y0_hbm, y1_hbm,
                     i_v, rows_v, sem):
    wid = lax.axis_index("s") * 2 + lax.axis_index("c")
    base = wid * TPW
    pltpu.sync_copy(s0_hbm.at[pl.ds(base, TPW)], i_v)
    pltpu.async_copy(os_hbm.at[i_v], rows_v, sem).wait()
    pltpu.sync_copy(rows_v, y0_hbm.at[pl.ds(base, TPW)])
    pltpu.sync_copy(s1_hbm.at[pl.ds(base, TPW)], i_v)
    pltpu.async_copy(os_hbm.at[i_v], rows_v, sem).wait()
    pltpu.sync_copy(rows_v, y1_hbm.at[pl.ds(base, TPW)])


def _weight_dmas(meta_ref, wg_hbm, w1_hbm, w2_hbm,
                 wg_scr, w1_scr, w2_scr, sems, run, buf):
    e = meta_ref[2, jnp.minimum(run, NUM_EXPERTS - 1)]
    return [
        pltpu.make_async_copy(wg_hbm.at[e], wg_scr.at[buf], sems.at[0, buf]),
        pltpu.make_async_copy(w1_hbm.at[e], w1_scr.at[buf], sems.at[1, buf]),
        pltpu.make_async_copy(w2_hbm.at[e], w2_scr.at[buf], sems.at[2, buf]),
    ]


def _group_mm_body(meta_ref, zs_ref, wg_hbm, w1_hbm, w2_hbm, out_ref,
                   wg_scr, w1_scr, w2_scr, sems):
    i = pl.program_id(0)
    run = meta_ref[1, i]
    run_prev = meta_ref[1, jnp.maximum(i - 1, 0)]
    last_run = meta_ref[1, NT - 1]
    is_run_start = jnp.logical_or(i == 0, run != run_prev)
    buf = jnp.remainder(run, 2)

    @pl.when(i == 0)
    def _():
        for d in _weight_dmas(meta_ref, wg_hbm, w1_hbm, w2_hbm,
                              wg_scr, w1_scr, w2_scr, sems, run, buf):
            d.start()

    @pl.when(is_run_start)
    def _():
        for d in _weight_dmas(meta_ref, wg_hbm, w1_hbm, w2_hbm,
                              wg_scr, w1_scr, w2_scr, sems, run, buf):
            d.wait()

    @pl.when(jnp.logical_and(is_run_start, run < last_run))
    def _():
        for d in _weight_dmas(meta_ref, wg_hbm, w1_hbm, w2_hbm,
                              wg_scr, w1_scr, w2_scr, sems,
                              run + 1, jnp.remainder(run + 1, 2)):
            d.start()

    z = zs_ref[...].astype(jnp.bfloat16)
    g = jnp.dot(z, wg_scr[buf].astype(jnp.bfloat16),
                preferred_element_type=jnp.float32)
    a = jnp.dot(z, w1_scr[buf].astype(jnp.bfloat16),
                preferred_element_type=jnp.float32)
    h = (g / (1.0 + jnp.exp(-g))) * a
    out_ref[...] = jnp.dot(h.astype(jnp.bfloat16),
                           w2_scr[buf].astype(jnp.bfloat16),
                           preferred_element_type=jnp.float32)


def _combine_up_body(y0_ref, y1_ref, g0_ref, g1_ref, wu_ref, y_ref):
    y_lat = g0_ref[...] * y0_ref[...] + g1_ref[...] * y1_ref[...]
    y_ref[...] = jnp.dot(y_lat.astype(jnp.bfloat16),
                         wu_ref[...].astype(jnp.bfloat16),
                         preferred_element_type=jnp.float32)


@jax.jit
def kernel(x, Wr, Wd, Wu, w1, wg, w2):
    B, S, D = x.shape
    N = B * S
    t = x.reshape(N, D)
    n_tiles = N // TILE

    z, topi0, topi1, gates0, gates1 = pl.pallas_call(
        _router_z_body,
        grid=(N // TILE_A,),
        in_specs=[
            pl.BlockSpec((TILE_A, D_MODEL), lambda i: (i, 0)),
            pl.BlockSpec((D_MODEL, NUM_EXPERTS), lambda i: (0, 0)),
            pl.BlockSpec((D_MODEL, D_LATENT), lambda i: (0, 0)),
        ],
        out_specs=[
            pl.BlockSpec((TILE_A, D_LATENT), lambda i: (i, 0)),
            pl.BlockSpec((TILE_A, 1), lambda i: (i, 0)),
            pl.BlockSpec((TILE_A, 1), lambda i: (i, 0)),
            pl.BlockSpec((TILE_A, 1), lambda i: (i, 0)),
            pl.BlockSpec((TILE_A, 1), lambda i: (i, 0)),
        ],
        out_shape=[
            jax.ShapeDtypeStruct((N, D_LATENT), jnp.float32),
            jax.ShapeDtypeStruct((N, 1), jnp.int32),
            jax.ShapeDtypeStruct((N, 1), jnp.int32),
            jax.ShapeDtypeStruct((N, 1), jnp.float32),
            jax.ShapeDtypeStruct((N, 1), jnp.float32),
        ],
    )(t, Wr, Wd)

    slot0m, slot1m, tile_eid8 = pl.pallas_call(
        _metadata_body,
        grid=(1,),
        in_specs=[
            pl.BlockSpec((_MR // 2, _MC), lambda i: (0, 0)),
            pl.BlockSpec((_MR // 2, _MC), lambda i: (0, 0)),
        ],
        out_specs=[
            pl.BlockSpec((_MR // 2, _MC), lambda i: (0, 0)),
            pl.BlockSpec((_MR // 2, _MC), lambda i: (0, 0)),
            pl.BlockSpec((8, 128), lambda i: (0, 0)),
        ],
        out_shape=[
            jax.ShapeDtypeStruct((_MR // 2, _MC), jnp.int32),
            jax.ShapeDtypeStruct((_MR // 2, _MC), jnp.int32),
            jax.ShapeDtypeStruct((8, 128), jnp.int32),
        ],
    )(topi0.reshape(_MR // 2, _MC), topi1.reshape(_MR // 2, _MC))

    slot0 = slot0m.reshape(N)
    slot1 = slot1m.reshape(N)

    sc_mesh = plsc.VectorSubcoreMesh(core_axis_name="c", subcore_axis_name="s")

    z_sorted = pl.kernel(
        _sc_dispatch_body,
        out_type=jax.ShapeDtypeStruct((S_PAD, D_LATENT), jnp.float32),
        mesh=sc_mesh,
        scratch_types=[
            pltpu.VMEM((TPW,), jnp.int32),
            pltpu.VMEM((TPW,), jnp.int32),
            pltpu.VMEM((TPW, D_LATENT), jnp.float32),
            pltpu.SemaphoreType.DMA,
        ],
    )(z, slot0, slot1)

    out_sorted = pl.pallas_call(
        _group_mm_body,
        grid_spec=pltpu.PrefetchScalarGridSpec(
            num_scalar_prefetch=1,
            grid=(NT,),
            in_specs=[
                pl.BlockSpec((TILE_G, D_LATENT), lambda i, eids: (i, 0)),
                pl.BlockSpec(memory_space=pl.ANY),
                pl.BlockSpec(memory_space=pl.ANY),
                pl.BlockSpec(memory_space=pl.ANY),
            ],
            out_specs=pl.BlockSpec((TILE_G, D_LATENT), lambda i, eids: (i, 0)),
            scratch_shapes=[
                pltpu.VMEM((2, D_LATENT, D_HIDDEN), jnp.float32),
                pltpu.VMEM((2, D_LATENT, D_HIDDEN), jnp.float32),
                pltpu.VMEM((2, D_HIDDEN, D_LATENT), jnp.float32),
                pltpu.SemaphoreType.DMA((3, 2)),
            ],
        ),
        out_shape=jax.ShapeDtypeStruct((S_PAD, D_LATENT), jnp.float32),
    )(tile_eid8, z_sorted, wg, w1, w2)

    y0, y1 = pl.kernel(
        _sc_combine_body,
        out_type=[
            jax.ShapeDtypeStruct((N, D_LATENT), jnp.float32),
            jax.ShapeDtypeStruct((N, D_LATENT), jnp.float32),
        ],
        mesh=sc_mesh,
        scratch_types=[
            pltpu.VMEM((TPW,), jnp.int32),
            pltpu.VMEM((TPW, D_LATENT), jnp.float32),
            pltpu.SemaphoreType.DMA,
        ],
    )(out_sorted, slot0, slot1)

    y = pl.pallas_call(
        _combine_up_body,
        grid=(N // TILE_E,),
        in_specs=[
            pl.BlockSpec((TILE_E, D_LATENT), lambda i: (i, 0)),
            pl.BlockSpec((TILE_E, D_LATENT), lambda i: (i, 0)),
            pl.BlockSpec((TILE_E, 1), lambda i: (i, 0)),
            pl.BlockSpec((TILE_E, 1), lambda i: (i, 0)),
            pl.BlockSpec((D_LATENT, D_MODEL), lambda i: (0, 0)),
        ],
        out_specs=pl.BlockSpec((TILE_E, D_MODEL), lambda i: (i, 0)),
        out_shape=jax.ShapeDtypeStruct((N, D_MODEL), jnp.float32),
    )(y0, y1, gates0, gates1, Wu)

    return y.reshape(B, S, D)


# speedup vs baseline: 1.0905x; 1.0196x over previous
import functools

import jax
import jax.numpy as jnp
from jax import lax
from jax.experimental import pallas as pl
from jax.experimental.pallas import tpu as pltpu
from jax.experimental.pallas import tpu_sc as plsc

D_MODEL = 2048
D_LATENT = 512
D_HIDDEN = 1024
NUM_EXPERTS = 8
TOP_K = 2
N_TOKENS = 4096
TILE = 256
TILE_A = 1024
TILE_E = 512
TILE_G = 256
NT = N_TOKENS * TOP_K // TILE_G + NUM_EXPERTS - 1
S_PAD = NT * TILE_G

_NEG_BIG = -3.0e38


def _col_to_rows(v):
    T = v.shape[0]
    n0 = lax.broadcasted_iota(jnp.int32, (T, 128), 0)
    c1 = lax.broadcasted_iota(jnp.int32, (T, 128), 1)
    b1 = (n0 % 128 == c1).astype(jnp.float32)
    r0 = lax.broadcasted_iota(jnp.int32, (T // 128, T), 0)
    n1 = lax.broadcasted_iota(jnp.int32, (T // 128, T), 1)
    a1 = (n1 // 128 == r0).astype(jnp.float32)
    return jnp.dot(a1, v * b1, preferred_element_type=jnp.float32)


def _router_z_body(x_ref, wr_ref, wd_ref, z_ref,
                   topi0_ref, topi1_ref, gates0_ref, gates1_ref):
    x = x_ref[0]
    logits = jnp.dot(x, wr_ref[...], preferred_element_type=jnp.float32)
    col = lax.broadcasted_iota(jnp.int32, logits.shape, 1)
    m1 = jnp.max(logits, axis=-1, keepdims=True)
    i1 = jnp.min(jnp.where(logits == m1, col, NUM_EXPERTS), axis=-1, keepdims=True)
    masked = jnp.where(col == i1, _NEG_BIG, logits)
    m2 = jnp.max(masked, axis=-1, keepdims=True)
    i2 = jnp.min(jnp.where(masked == m2, col, NUM_EXPERTS), axis=-1, keepdims=True)
    g0 = 1.0 / (1.0 + jnp.exp(m2 - m1))
    topi0_ref[...] = _col_to_rows(i1.astype(jnp.float32)).astype(jnp.int32)
    topi1_ref[...] = _col_to_rows(i2.astype(jnp.float32)).astype(jnp.int32)
    gates0_ref[...] = g0
    gates1_ref[...] = 1.0 - g0
    z_ref[...] = jnp.dot(x.astype(jnp.bfloat16),
                         wd_ref[...].astype(jnp.bfloat16),
                         preferred_element_type=jnp.float32)


_MR = 64
_MC = 128


def _metadata_body(eid0_ref, eid1_ref, slot0_ref, slot1_ref, tile_eid_ref):
    eid = jnp.concatenate([eid0_ref[...], eid1_ref[...]], axis=0)
    row = lax.broadcasted_iota(jnp.int32, (_MR, _MR), 0)
    rowp = lax.broadcasted_iota(jnp.int32, (_MR, _MR), 1)
    l_strict = (rowp < row).astype(jnp.float32)
    colp = lax.broadcasted_iota(jnp.int32, (_MC, _MC), 0)
    colc = lax.broadcasted_iota(jnp.int32, (_MC, _MC), 1)
    u_strict = (colp < colc).astype(jnp.float32)
    ones_c = jnp.ones((_MC, _MC), jnp.float32)

    slot_f = jnp.zeros((_MR, _MC), jnp.float32)
    off = 0
    starts = []
    ntiles = []
    for e in range(NUM_EXPERTS):
        m_e = (eid == e).astype(jnp.float32)
        prior_rows = jnp.dot(
            jnp.dot(l_strict, m_e, preferred_element_type=jnp.float32),
            ones_c, preferred_element_type=jnp.float32)
        in_row = jnp.dot(m_e, u_strict, preferred_element_type=jnp.float32)
        rank = prior_rows + in_row
        c_e = jnp.sum(m_e).astype(jnp.int32)
        off_f = off.astype(jnp.float32) if e > 0 else jnp.float32(0.0)
        slot_f = slot_f + m_e * (off_f + rank)
        nt_e = (c_e + (TILE_G - 1)) // TILE_G
        start_e = (off // TILE_G) if e > 0 else jnp.int32(0)
        starts.append(start_e)
        ntiles.append(nt_e)
        off = (off if e > 0 else jnp.int32(0)) + nt_e * TILE_G
    slot = slot_f.astype(jnp.int32)
    slot0_ref[...] = slot[:_MR // 2, :]
    slot1_ref[...] = slot[_MR // 2:, :]

    tcol = lax.broadcasted_iota(jnp.int32, (8, 128), 1)
    acc = jnp.zeros((8, 128), jnp.int32)
    acc_run = jnp.zeros((8, 128), jnp.int32)
    acc_reid = jnp.zeros((8, 128), jnp.int32)
    prefix_present = jnp.int32(0)
    for e in range(NUM_EXPERTS):
        hit = ((tcol >= starts[e]) & (tcol < starts[e] + ntiles[e])
               ).astype(jnp.int32)
        acc = acc + e * hit
        acc_run = acc_run + prefix_present * hit
        present = (ntiles[e] > 0).astype(jnp.int32)
        acc_reid = acc_reid + e * ((tcol == prefix_present).astype(jnp.int32)
                                   * present)
        prefix_present = prefix_present + present
    total_used = off // TILE_G
    acc_run = acc_run + (prefix_present - 1) * (tcol >= total_used).astype(
        jnp.int32)
    tile_eid_ref[...] = jnp.concatenate(
        [acc[0:1, :], acc_run[0:1, :], acc_reid[0:1, :], acc[0:5, :]], axis=0)


TPW = N_TOKENS // 32


def _sc_dispatch_body(z_hbm, s0_hbm, s1_hbm, zs_hbm, i0_v, i1_v, rows_v, sem):
    wid = lax.axis_index("s") * 2 + lax.axis_index("c")
    base = wid * TPW
    pltpu.sync_copy(s0_hbm.at[pl.ds(base, TPW)], i0_v)
    pltpu.sync_copy(s1_hbm.at[pl.ds(base, TPW)], i1_v)
    pltpu.sync_copy(z_hbm.at[pl.ds(base, TPW)], rows_v)
    pltpu.async_copy(rows_v, zs_hbm.at[i0_v], sem).wait()
    pltpu.async_copy(rows_v, zs_hbm.at[i1_v], sem).wait()


def _sc_combine_body(os_hbm, s0_hbm, s1_hbm, y0_hbm, y1_hbm,
                     i_v, rows_v, sem):
    wid = lax.axis_index("s") * 2 + lax.axis_index("c")
    base = wid * TPW
    pltpu.sync_copy(s0_hbm.at[pl.ds(base, TPW)], i_v)
    pltpu.async_copy(os_hbm.at[i_v], rows_v, sem).wait()
    pltpu.sync_copy(rows_v, y0_hbm.at[pl.ds(base, TPW)])
    pltpu.sync_copy(s1_hbm.at[pl.ds(base, TPW)], i_v)
    pltpu.async_copy(os_hbm.at[i_v], rows_v, sem).wait()
    pltpu.sync_copy(rows_v, y1_hbm.at[pl.ds(base, TPW)])


def _weight_dmas(meta_ref, wg_hbm, w1_hbm, w2_hbm,
                 wg_scr, w1_scr, w2_scr, sems, run, buf):
    e = meta_ref[2, jnp.minimum(run, NUM_EXPERTS - 1)]
    return [
        pltpu.make_async_copy(wg_hbm.at[e], wg_scr.at[buf], sems.at[0, buf]),
        pltpu.make_async_copy(w1_hbm.at[e], w1_scr.at[buf], sems.at[1, buf]),
        pltpu.make_async_copy(w2_hbm.at[e], w2_scr.at[buf], sems.at[2, buf]),
    ]


def _group_mm_body(meta_ref, zs_ref, wg_hbm, w1_hbm, w2_hbm, out_ref,
                   wg_scr, w1_scr, w2_scr, sems):
    i = pl.program_id(0)
    run = meta_ref[1, i]
    run_prev = meta_ref[1, jnp.maximum(i - 1, 0)]
    last_run = meta_ref[1, NT - 1]
    is_run_start = jnp.logical_or(i == 0, run != run_prev)
    buf = jnp.remainder(run, 2)

    @pl.when(i == 0)
    def _():
        for d in _weight_dmas(meta_ref, wg_hbm, w1_hbm, w2_hbm,
                              wg_scr, w1_scr, w2_scr, sems, run, buf):
            d.start()

    @pl.when(is_run_start)
    def _():
        for d in _weight_dmas(meta_ref, wg_hbm, w1_hbm, w2_hbm,
                              wg_scr, w1_scr, w2_scr, sems, run, buf):
            d.wait()

    @pl.when(jnp.logical_and(is_run_start, run < last_run))
    def _():
        for d in _weight_dmas(meta_ref, wg_hbm, w1_hbm, w2_hbm,
                              wg_scr, w1_scr, w2_scr, sems,
                              run + 1, jnp.remainder(run + 1, 2)):
            d.start()

    z = zs_ref[...].astype(jnp.bfloat16)
    g = jnp.dot(z, wg_scr[buf].astype(jnp.bfloat16),
                preferred_element_type=jnp.float32)
    a = jnp.dot(z, w1_scr[buf].astype(jnp.bfloat16),
                preferred_element_type=jnp.float32)
    h = (g / (1.0 + jnp.exp(-g))) * a
    out_ref[...] = jnp.dot(h.astype(jnp.bfloat16),
                           w2_scr[buf].astype(jnp.bfloat16),
                           preferred_element_type=jnp.float32)


def _combine_up_body(y0_ref, y1_ref, g0_ref, g1_ref, wu_ref, y_ref):
    y_lat = g0_ref[...] * y0_ref[...] + g1_ref[...] * y1_ref[...]
    y_ref[...] = jnp.dot(y_lat.astype(jnp.bfloat16),
                         wu_ref[...].astype(jnp.bfloat16),
                         preferred_element_type=jnp.float32)


@jax.jit
def kernel(x, Wr, Wd, Wu, w1, wg, w2):
    B, S, D = x.shape
    N = B * S
    t = x.reshape(N, D)
    n_tiles = N // TILE

    z, topi0, topi1, gates0, gates1 = pl.pallas_call(
        _router_z_body,
        grid=(N // TILE_A,),
        in_specs=[
            pl.BlockSpec((1, TILE_A, D_MODEL), lambda i: (0, i, 0)),
            pl.BlockSpec((D_MODEL, NUM_EXPERTS), lambda i: (0, 0)),
            pl.BlockSpec((D_MODEL, D_LATENT), lambda i: (0, 0)),
        ],
        out_specs=[
            pl.BlockSpec((TILE_A, D_LATENT), lambda i: (i, 0)),
            pl.BlockSpec((TILE_A // 128, 128), lambda i: (i, 0)),
            pl.BlockSpec((TILE_A // 128, 128), lambda i: (i, 0)),
            pl.BlockSpec((TILE_A, 1), lambda i: (i, 0)),
            pl.BlockSpec((TILE_A, 1), lambda i: (i, 0)),
        ],
        out_shape=[
            jax.ShapeDtypeStruct((N, D_LATENT), jnp.float32),
            jax.ShapeDtypeStruct((N // 128, 128), jnp.int32),
            jax.ShapeDtypeStruct((N // 128, 128), jnp.int32),
            jax.ShapeDtypeStruct((N, 1), jnp.float32),
            jax.ShapeDtypeStruct((N, 1), jnp.float32),
        ],
    )(x, Wr, Wd)

    slot0m, slot1m, tile_eid8 = pl.pallas_call(
        _metadata_body,
        grid=(1,),
        in_specs=[
            pl.BlockSpec((_MR // 2, _MC), lambda i: (0, 0)),
            pl.BlockSpec((_MR // 2, _MC), lambda i: (0, 0)),
        ],
        out_specs=[
            pl.BlockSpec((_MR // 2, _MC), lambda i: (0, 0)),
            pl.BlockSpec((_MR // 2, _MC), lambda i: (0, 0)),
            pl.BlockSpec((8, 128), lambda i: (0, 0)),
        ],
        out_shape=[
            jax.ShapeDtypeStruct((_MR // 2, _MC), jnp.int32),
            jax.ShapeDtypeStruct((_MR // 2, _MC), jnp.int32),
            jax.ShapeDtypeStruct((8, 128), jnp.int32),
        ],
    )(topi0, topi1)

    slot0 = slot0m.reshape(N)
    slot1 = slot1m.reshape(N)

    sc_mesh = plsc.VectorSubcoreMesh(core_axis_name="c", subcore_axis_name="s")

    z_sorted = pl.kernel(
        _sc_dispatch_body,
        out_type=jax.ShapeDtypeStruct((S_PAD, D_LATENT), jnp.float32),
        mesh=sc_mesh,
        scratch_types=[
            pltpu.VMEM((TPW,), jnp.int32),
            pltpu.VMEM((TPW,), jnp.int32),
            pltpu.VMEM((TPW, D_LATENT), jnp.float32),
            pltpu.SemaphoreType.DMA,
        ],
    )(z, slot0, slot1)

    out_sorted = pl.pallas_call(
        _group_mm_body,
        grid_spec=pltpu.PrefetchScalarGridSpec(
            num_scalar_prefetch=1,
            grid=(NT,),
            in_specs=[
                pl.BlockSpec((TILE_G, D_LATENT), lambda i, eids: (i, 0)),
                pl.BlockSpec(memory_space=pl.ANY),
                pl.BlockSpec(memory_space=pl.ANY),
                pl.BlockSpec(memory_space=pl.ANY),
            ],
            out_specs=pl.BlockSpec((TILE_G, D_LATENT), lambda i, eids: (i, 0)),
            scratch_shapes=[
                pltpu.VMEM((2, D_LATENT, D_HIDDEN), jnp.float32),
                pltpu.VMEM((2, D_LATENT, D_HIDDEN), jnp.float32),
                pltpu.VMEM((2, D_HIDDEN, D_LATENT), jnp.float32),
                pltpu.SemaphoreType.DMA((3, 2)),
            ],
        ),
        out_shape=jax.ShapeDtypeStruct((S_PAD, D_LATENT), jnp.float32),
    )(tile_eid8, z_sorted, wg, w1, w2)

    y0, y1 = pl.kernel(
        _sc_combine_body,
        out_type=[
            jax.ShapeDtypeStruct((N, D_LATENT), jnp.float32),
            jax.ShapeDtypeStruct((N, D_LATENT), jnp.float32),
        ],
        mesh=sc_mesh,
        scratch_types=[
            pltpu.VMEM((TPW,), jnp.int32),
            pltpu.VMEM((TPW, D_LATENT), jnp.float32),
            pltpu.SemaphoreType.DMA,
        ],
    )(out_sorted, slot0, slot1)

    y = pl.pallas_call(
        _combine_up_body,
        grid=(N // TILE_E,),
        in_specs=[
            pl.BlockSpec((TILE_E, D_LATENT), lambda i: (i, 0)),
            pl.BlockSpec((TILE_E, D_LATENT), lambda i: (i, 0)),
            pl.BlockSpec((TILE_E, 1), lambda i: (i, 0)),
            pl.BlockSpec((TILE_E, 1), lambda i: (i, 0)),
            pl.BlockSpec((D_LATENT, D_MODEL), lambda i: (0, 0)),
        ],
        out_specs=pl.BlockSpec((TILE_E, D_MODEL), lambda i: (i, 0)),
        out_shape=jax.ShapeDtypeStruct((N, D_MODEL), jnp.float32),
    )(y0, y1, gates0, gates1, Wu)

    return y.reshape(B, S, D)
